# baseline (speedup 1.0000x reference)
import functools

import jax
import jax.numpy as jnp
from jax import lax
from jax.experimental import pallas as pl
from jax.experimental.pallas import tpu as pltpu
from jax.experimental.pallas import tpu_sc as plsc

NUM_CORES = 2
NUM_SUBCORES = 16
LANES = 16
NW = NUM_CORES * NUM_SUBCORES
BATCH = 16384
EMB = 64
B_PER_W = BATCH // NW
IDX_CHUNK = 128


def _svd_body(u_hbm, v_hbm, ue_hbm, ub_hbm, ie_hbm, ib_hbm, mean_hbm,
              out_hbm,
              uidx_v, vidx_v, urows_v, irows_v, ubias_v, ibias_v,
              mean_v, out_v, sem):
    wid = lax.axis_index("s") * NUM_CORES + lax.axis_index("c")
    base = wid * B_PER_W

    pltpu.sync_copy(u_hbm.at[pl.ds(base, B_PER_W)], uidx_v)
    pltpu.sync_copy(v_hbm.at[pl.ds(base, B_PER_W)], vidx_v)
    pltpu.sync_copy(mean_hbm, mean_v)

    c0 = pltpu.async_copy(ue_hbm.at[uidx_v], urows_v, sem)
    c1 = pltpu.async_copy(ie_hbm.at[vidx_v], irows_v, sem)
    c2 = pltpu.async_copy(ub_hbm.at[uidx_v], ubias_v, sem)
    c3 = pltpu.async_copy(ib_hbm.at[vidx_v], ibias_v, sem)
    c0.wait()
    c1.wait()
    c2.wait()
    c3.wait()

    zeros_i = jnp.zeros((LANES,), jnp.int32)
    mean_s = mean_v[...]
    zeros_f = jnp.zeros((LANES,), jnp.float32)
    lane_iota = lax.iota(jnp.int32, LANES)

    def group(g, carry):
        rows = g * LANES + lane_iota
        acc = zeros_f
        for e in range(EMB):
            col = jnp.full((LANES,), e, jnp.int32)
            uu = plsc.load_gather(urows_v, [rows, col])
            ii = plsc.load_gather(irows_v, [rows, col])
            acc = acc + uu * ii
        bu = plsc.load_gather(ubias_v, [rows])
        bi = plsc.load_gather(ibias_v, [rows])
        out_v[pl.ds(g * LANES, LANES)] = acc + bu + bi + mean_s
        return carry

    lax.fori_loop(0, B_PER_W // LANES, group, 0)

    pltpu.sync_copy(out_v, out_hbm.at[pl.ds(base, B_PER_W)])


@jax.jit
def _svd_sc(u, v, user_emb, user_emb_bias, item_emb, item_emb_bias, mean):
    mesh = plsc.VectorSubcoreMesh(core_axis_name="c", subcore_axis_name="s",
                                  num_cores=NUM_CORES,
                                  num_subcores=NUM_SUBCORES)
    run = pl.kernel(
        _svd_body,
        out_type=jax.ShapeDtypeStruct((BATCH,), jnp.float32),
        mesh=mesh,
        compiler_params=pltpu.CompilerParams(needs_layout_passes=False,
                                             use_tc_tiling_on_sc=False),
        scratch_types=[
            pltpu.VMEM((B_PER_W,), jnp.int32),
            pltpu.VMEM((B_PER_W,), jnp.int32),
            pltpu.VMEM((B_PER_W, EMB), jnp.float32),
            pltpu.VMEM((B_PER_W, EMB), jnp.float32),
            pltpu.VMEM((B_PER_W,), jnp.float32),
            pltpu.VMEM((B_PER_W,), jnp.float32),
            pltpu.VMEM((LANES,), jnp.float32),
            pltpu.VMEM((B_PER_W,), jnp.float32),
            pltpu.SemaphoreType.DMA,
        ],
    )
    return run(u, v, user_emb, user_emb_bias, item_emb, item_emb_bias, mean)


def kernel(u, v, user_emb, user_emb_bias, item_emb, item_emb_bias, mean):
    return _svd_sc(u.astype(jnp.int32), v.astype(jnp.int32), user_emb,
                   user_emb_bias.reshape(-1), item_emb,
                   item_emb_bias.reshape(-1),
                   jnp.tile(mean.astype(jnp.float32), LANES))
